# baseline (speedup 1.0000x reference)
import jax
import jax.numpy as jnp
from jax.experimental import pallas as pl
from jax.experimental.pallas import tpu as pltpu

N_RAW = 5000
N_PAD = 5120
BLK = 256
NB = N_PAD // BLK
OUT_K = 300
OUT_PAD = 304
IOU_THR = 0.5
SCORE_THR = 0.05
FP_CHUNK = 6

_HI = jax.lax.Precision.HIGHEST
_f32 = jnp.float32


def _nms_kernel(x1_ref, y1_ref, x2_ref, y2_ref, s_ref,
                cx1_ref, cy1_ref, cx2_ref, cy2_ref,
                obox_ref, os_ref, alive_ref, dest_ref):
    s = s_ref[...]

    sub = jax.lax.broadcasted_iota(jnp.int32, (BLK, BLK), 0)
    lan = jax.lax.broadcasted_iota(jnp.int32, (BLK, BLK), 1)
    eye = jnp.where(sub == lan, 1.0, 0.0).astype(_f32)
    lti = jnp.where(sub <= lan, 1.0, 0.0).astype(_f32)

    def tr(row):
        return jax.lax.dot_general(eye, row, (((1,), (1,)), ((), ())),
                                   precision=_HI)

    alive_ref[...] = jnp.where(s > SCORE_THR, 1.0, 0.0).astype(_f32)

    for k in range(NB):
        lo = k * BLK
        hi = lo + BLK
        bx1 = x1_ref[0:1, lo:hi]
        by1 = y1_ref[0:1, lo:hi]
        bx2 = x2_ref[0:1, lo:hi]
        by2 = y2_ref[0:1, lo:hi]
        cx1 = cx1_ref[lo:hi, 0:1]
        cy1 = cy1_ref[lo:hi, 0:1]
        cx2 = cx2_ref[lo:hi, 0:1]
        cy2 = cy2_ref[lo:hi, 0:1]
        balive = alive_ref[0:1, lo:hi]
        calive = tr(balive)

        areac = (cx2 - cx1) * (cy2 - cy1)
        arear = (bx2 - bx1) * (by2 - by1)

        ix1 = jnp.maximum(cx1, bx1)
        iy1 = jnp.maximum(cy1, by1)
        ix2 = jnp.minimum(cx2, bx2)
        iy2 = jnp.minimum(cy2, by2)
        iw = jnp.maximum(ix2 - ix1, 0.0)
        ih = jnp.maximum(iy2 - iy1, 0.0)
        inter = iw * ih
        union = areac + arear - inter
        iou = inter / jnp.maximum(union, 1e-9)
        sl = jnp.where((iou > IOU_THR) & (lan < sub), 1.0, 0.0).astype(_f32)

        def fp_cond(c):
            return c[1]

        def fp_body(c, calive=calive, sl=sl):
            keep0, _ = c
            keep = keep0
            for _ in range(FP_CHUNK):
                supp = jax.lax.dot_general(sl, keep, (((1,), (0,)), ((), ())))
                keep = calive * jnp.where(supp < 0.5, 1.0, 0.0)
            changed = jnp.sum(jnp.abs(keep - keep0)) > 0.0
            return (keep, changed)

        keepc, _ = jax.lax.while_loop(fp_cond, fp_body,
                                      (calive, jnp.array(True)))

        keeprow = jax.lax.dot_general(keepc, eye, (((0,), (0,)), ((), ())),
                                      precision=_HI)
        alive_ref[0:1, lo:hi] = keeprow

        if hi < N_PAD:
            kx2 = jnp.where(keepc > 0.5, cx2, -3e38)
            tx1g = x1_ref[0:1, hi:N_PAD]
            ty1g = y1_ref[0:1, hi:N_PAD]
            tx2g = x2_ref[0:1, hi:N_PAD]
            ty2g = y2_ref[0:1, hi:N_PAD]
            tarea = (tx2g - tx1g) * (ty2g - ty1g)
            tx1 = jnp.maximum(cx1, tx1g)
            ty1 = jnp.maximum(cy1, ty1g)
            tx2 = jnp.minimum(kx2, tx2g)
            ty2 = jnp.minimum(cy2, ty2g)
            tw = jnp.maximum(tx2 - tx1, 0.0)
            th = jnp.maximum(ty2 - ty1, 0.0)
            tinter = tw * th
            tunion = areac + tarea - tinter
            tiou = tinter / jnp.maximum(tunion, 1e-9)
            supp = jnp.any(tiou > IOU_THR, axis=0, keepdims=True)
            alive_ref[0:1, hi:N_PAD] = (alive_ref[0:1, hi:N_PAD]
                                        * jnp.where(supp, 0.0, 1.0))

    alive = alive_ref[...]
    total_k = jnp.sum(alive)

    koff = jnp.float32(0.0)
    soff = jnp.float32(0.0)
    for k in range(NB):
        lo = k * BLK
        hi = lo + BLK
        row = alive_ref[0:1, lo:hi]
        kcum = jax.lax.dot_general(row, lti, (((1,), (0,)), ((), ())))
        nrow = 1.0 - row
        scum = jax.lax.dot_general(nrow, lti, (((1,), (0,)), ((), ())))
        dest_ref[0:1, lo:hi] = jnp.where(row > 0.5, koff + kcum - 1.0,
                                         total_k + soff + scum - 1.0)
        koff = koff + jnp.sum(row)
        soff = soff + jnp.sum(nrow)

    dest = dest_ref[...].astype(jnp.int32)
    tsub = jax.lax.broadcasted_iota(jnp.int32, (OUT_PAD, N_PAD), 0)
    m = jnp.where(dest == tsub, 1.0, 0.0).astype(_f32)

    def sel(row):
        return jax.lax.dot_general(m, row, (((1,), (1,)), ((), ())),
                                   precision=_HI)

    obox = jnp.concatenate([sel(x1_ref[...]), sel(y1_ref[...]),
                            sel(x2_ref[...]), sel(y2_ref[...])], axis=1)
    obox_ref[...] = obox
    smask = jnp.where(alive > 0.5, s, -1.0)
    os_ref[...] = jax.lax.dot_general(smask, m, (((1,), (1,)), ((), ())),
                                      precision=_HI)


def _run_nms(x1, y1, x2, y2, s, cx1, cy1, cx2, cy2):
    return pl.pallas_call(
        _nms_kernel,
        out_shape=[
            jax.ShapeDtypeStruct((OUT_PAD, 4), _f32),
            jax.ShapeDtypeStruct((1, OUT_PAD), _f32),
        ],
        scratch_shapes=[
            pltpu.VMEM((1, N_PAD), _f32),
            pltpu.VMEM((1, N_PAD), _f32),
        ],
    )(x1, y1, x2, y2, s, cx1, cy1, cx2, cy2)


def kernel(boxes, scores):
    order = jnp.argsort(-scores)
    b = boxes[order]
    s = scores[order]
    pad = N_PAD - N_RAW
    bp = jnp.concatenate([b, jnp.zeros((pad, 4), _f32)], axis=0)
    sp = jnp.concatenate([s, jnp.full((pad,), -3e38, _f32)], axis=0)
    x1 = bp[:, 0].reshape(1, N_PAD)
    y1 = bp[:, 1].reshape(1, N_PAD)
    x2 = bp[:, 2].reshape(1, N_PAD)
    y2 = bp[:, 3].reshape(1, N_PAD)
    cx1 = bp[:, 0].reshape(N_PAD, 1)
    cy1 = bp[:, 1].reshape(N_PAD, 1)
    cx2 = bp[:, 2].reshape(N_PAD, 1)
    cy2 = bp[:, 3].reshape(N_PAD, 1)
    sp = sp.reshape(1, N_PAD)
    obox, ts = _run_nms(x1, y1, x2, y2, sp, cx1, cy1, cx2, cy2)
    return obox[:OUT_K], ts[0, :OUT_K]


# speedup vs baseline: 1.1101x; 1.1101x over previous
import jax
import jax.numpy as jnp
from jax.experimental import pallas as pl
from jax.experimental.pallas import tpu as pltpu

N_RAW = 5000
N_PAD = 5120
BLK = 128
NB = N_PAD // BLK
OUT_K = 300
OUT_PAD = 304
IOU_THR = 0.5
SCORE_THR = 0.05
FP_CHUNK = 4

_HI = jax.lax.Precision.HIGHEST
_f32 = jnp.float32


def _nms_kernel(x1_ref, y1_ref, x2_ref, y2_ref, s_ref,
                cx1_ref, cy1_ref, cx2_ref, cy2_ref,
                obox_ref, os_ref, alive_ref, dest_ref):
    s = s_ref[...]

    sub = jax.lax.broadcasted_iota(jnp.int32, (BLK, BLK), 0)
    lan = jax.lax.broadcasted_iota(jnp.int32, (BLK, BLK), 1)
    eye = jnp.where(sub == lan, 1.0, 0.0).astype(_f32)
    lti = jnp.where(sub <= lan, 1.0, 0.0).astype(_f32)

    def tr(row):
        return jax.lax.dot_general(eye, row, (((1,), (1,)), ((), ())),
                                   precision=_HI)

    alive_ref[...] = jnp.where(s > SCORE_THR, 1.0, 0.0).astype(_f32)

    for k in range(NB):
        lo = k * BLK
        hi = lo + BLK
        bx1 = x1_ref[0:1, lo:hi]
        by1 = y1_ref[0:1, lo:hi]
        bx2 = x2_ref[0:1, lo:hi]
        by2 = y2_ref[0:1, lo:hi]
        cx1 = cx1_ref[lo:hi, 0:1]
        cy1 = cy1_ref[lo:hi, 0:1]
        cx2 = cx2_ref[lo:hi, 0:1]
        cy2 = cy2_ref[lo:hi, 0:1]
        balive = alive_ref[0:1, lo:hi]
        calive = tr(balive)

        areac = (cx2 - cx1) * (cy2 - cy1)
        arear = (bx2 - bx1) * (by2 - by1)

        ix1 = jnp.maximum(cx1, bx1)
        iy1 = jnp.maximum(cy1, by1)
        ix2 = jnp.minimum(cx2, bx2)
        iy2 = jnp.minimum(cy2, by2)
        iw = jnp.maximum(ix2 - ix1, 0.0)
        ih = jnp.maximum(iy2 - iy1, 0.0)
        inter = iw * ih
        union = areac + arear - inter
        iou = inter / jnp.maximum(union, 1e-9)
        sl = jnp.where((iou > IOU_THR) & (lan < sub), 1.0, 0.0).astype(_f32)

        def fp_cond(c):
            return c[1]

        def fp_body(c, calive=calive, sl=sl):
            keep, _ = c
            for _ in range(FP_CHUNK):
                prev = keep
                supp = jax.lax.dot_general(sl, keep, (((1,), (0,)), ((), ())))
                keep = calive * jnp.where(supp < 0.5, 1.0, 0.0)
            changed = jnp.sum(jnp.abs(keep - prev)) > 0.0
            return (keep, changed)

        keepc, _ = jax.lax.while_loop(fp_cond, fp_body,
                                      (calive, jnp.array(True)))

        keeprow = jax.lax.dot_general(keepc, eye, (((0,), (0,)), ((), ())),
                                      precision=_HI)
        alive_ref[0:1, lo:hi] = keeprow

        if hi < N_PAD:
            kx2 = jnp.where(keepc > 0.5, cx2, -3e38)
            tx1g = x1_ref[0:1, hi:N_PAD]
            ty1g = y1_ref[0:1, hi:N_PAD]
            tx2g = x2_ref[0:1, hi:N_PAD]
            ty2g = y2_ref[0:1, hi:N_PAD]
            tarea = (tx2g - tx1g) * (ty2g - ty1g)
            tx1 = jnp.maximum(cx1, tx1g)
            ty1 = jnp.maximum(cy1, ty1g)
            tx2 = jnp.minimum(kx2, tx2g)
            ty2 = jnp.minimum(cy2, ty2g)
            tw = jnp.maximum(tx2 - tx1, 0.0)
            th = jnp.maximum(ty2 - ty1, 0.0)
            tinter = tw * th
            tunion = areac + tarea - tinter
            tiou = tinter / jnp.maximum(tunion, 1e-9)
            supp = jnp.any(tiou > IOU_THR, axis=0, keepdims=True)
            alive_ref[0:1, hi:N_PAD] = (alive_ref[0:1, hi:N_PAD]
                                        * jnp.where(supp, 0.0, 1.0))

    alive = alive_ref[...]
    total_k = jnp.sum(alive)

    koff = jnp.float32(0.0)
    soff = jnp.float32(0.0)
    for k in range(NB):
        lo = k * BLK
        hi = lo + BLK
        row = alive_ref[0:1, lo:hi]
        kcum = jax.lax.dot_general(row, lti, (((1,), (0,)), ((), ())))
        nrow = 1.0 - row
        scum = jax.lax.dot_general(nrow, lti, (((1,), (0,)), ((), ())))
        dest_ref[0:1, lo:hi] = jnp.where(row > 0.5, koff + kcum - 1.0,
                                         total_k + soff + scum - 1.0)
        koff = koff + jnp.sum(row)
        soff = soff + jnp.sum(nrow)

    dest = dest_ref[...].astype(jnp.int32)
    tsub = jax.lax.broadcasted_iota(jnp.int32, (OUT_PAD, N_PAD), 0)
    m = jnp.where(dest == tsub, 1.0, 0.0).astype(_f32)

    def sel(row):
        return jax.lax.dot_general(m, row, (((1,), (1,)), ((), ())),
                                   precision=_HI)

    obox = jnp.concatenate([sel(x1_ref[...]), sel(y1_ref[...]),
                            sel(x2_ref[...]), sel(y2_ref[...])], axis=1)
    obox_ref[...] = obox
    smask = jnp.where(alive > 0.5, s, -1.0)
    os_ref[...] = jax.lax.dot_general(smask, m, (((1,), (1,)), ((), ())),
                                      precision=_HI)


def _run_nms(x1, y1, x2, y2, s, cx1, cy1, cx2, cy2):
    return pl.pallas_call(
        _nms_kernel,
        out_shape=[
            jax.ShapeDtypeStruct((OUT_PAD, 4), _f32),
            jax.ShapeDtypeStruct((1, OUT_PAD), _f32),
        ],
        scratch_shapes=[
            pltpu.VMEM((1, N_PAD), _f32),
            pltpu.VMEM((1, N_PAD), _f32),
        ],
    )(x1, y1, x2, y2, s, cx1, cy1, cx2, cy2)


def kernel(boxes, scores):
    order = jnp.argsort(-scores)
    b = boxes[order]
    s = scores[order]
    pad = N_PAD - N_RAW
    bp = jnp.concatenate([b, jnp.zeros((pad, 4), _f32)], axis=0)
    sp = jnp.concatenate([s, jnp.full((pad,), -3e38, _f32)], axis=0)
    x1 = bp[:, 0].reshape(1, N_PAD)
    y1 = bp[:, 1].reshape(1, N_PAD)
    x2 = bp[:, 2].reshape(1, N_PAD)
    y2 = bp[:, 3].reshape(1, N_PAD)
    cx1 = bp[:, 0].reshape(N_PAD, 1)
    cy1 = bp[:, 1].reshape(N_PAD, 1)
    cx2 = bp[:, 2].reshape(N_PAD, 1)
    cy2 = bp[:, 3].reshape(N_PAD, 1)
    sp = sp.reshape(1, N_PAD)
    obox, ts = _run_nms(x1, y1, x2, y2, sp, cx1, cy1, cx2, cy2)
    return obox[:OUT_K], ts[0, :OUT_K]


# speedup vs baseline: 1.5475x; 1.3941x over previous
import jax
import jax.numpy as jnp
from jax.experimental import pallas as pl
from jax.experimental.pallas import tpu as pltpu

N_RAW = 5000
N_PAD = 5120
BLK = 128
NB = N_PAD // BLK
OUT_K = 300
OUT_PAD = 304
IOU_THR = 0.5
SCORE_THR = 0.05
FP_CHUNK = 4

_HI = jax.lax.Precision.HIGHEST
_f32 = jnp.float32


def _nms_kernel(x1_ref, y1_ref, x2_ref, y2_ref, s_ref,
                cx1_ref, cy1_ref, cx2_ref, cy2_ref,
                obox_ref, os_ref, alive_ref, dest_ref):
    s = s_ref[...]

    sub = jax.lax.broadcasted_iota(jnp.int32, (BLK, BLK), 0)
    lan = jax.lax.broadcasted_iota(jnp.int32, (BLK, BLK), 1)
    eye = jnp.where(sub == lan, 1.0, 0.0).astype(_f32)
    lti = jnp.where(sub <= lan, 1.0, 0.0).astype(_f32)

    def tr(row):
        return jax.lax.dot_general(eye, row, (((1,), (1,)), ((), ())),
                                   precision=_HI)

    alive_ref[...] = jnp.where(s > SCORE_THR, 1.0, 0.0).astype(_f32)

    for k in range(NB):
        lo = k * BLK
        hi = lo + BLK
        bx1 = x1_ref[0:1, lo:hi]
        by1 = y1_ref[0:1, lo:hi]
        bx2 = x2_ref[0:1, lo:hi]
        by2 = y2_ref[0:1, lo:hi]
        cx1 = cx1_ref[lo:hi, 0:1]
        cy1 = cy1_ref[lo:hi, 0:1]
        cx2 = cx2_ref[lo:hi, 0:1]
        cy2 = cy2_ref[lo:hi, 0:1]
        balive = alive_ref[0:1, lo:hi]
        calive = tr(balive)

        areac = (cx2 - cx1) * (cy2 - cy1)
        arear = (bx2 - bx1) * (by2 - by1)

        ix1 = jnp.maximum(cx1, bx1)
        iy1 = jnp.maximum(cy1, by1)
        ix2 = jnp.minimum(cx2, bx2)
        iy2 = jnp.minimum(cy2, by2)
        iw = jnp.maximum(ix2 - ix1, 0.0)
        ih = jnp.maximum(iy2 - iy1, 0.0)
        inter = iw * ih
        union = areac + arear - inter
        iou = inter / jnp.maximum(union, 1e-9)
        sl = jnp.where((iou > IOU_THR) & (lan < sub), 1.0, 0.0).astype(_f32)

        def fp_cond(c):
            return c[1]

        def fp_body(c, calive=calive, sl=sl):
            keep, _ = c
            for _ in range(FP_CHUNK):
                prev = keep
                supp = jax.lax.dot_general(sl, keep, (((1,), (0,)), ((), ())))
                keep = calive * jnp.where(supp < 0.5, 1.0, 0.0)
            changed = jnp.sum(jnp.abs(keep - prev)) > 0.0
            return (keep, changed)

        keepc, _ = jax.lax.while_loop(fp_cond, fp_body,
                                      (calive, jnp.array(True)))

        keeprow = jax.lax.dot_general(keepc, eye, (((0,), (0,)), ((), ())),
                                      precision=_HI)
        alive_ref[0:1, lo:hi] = keeprow

        if hi < N_PAD:
            kx2 = jnp.where(keepc > 0.5, cx2, -3e38)
            tx1g = x1_ref[0:1, hi:N_PAD]
            ty1g = y1_ref[0:1, hi:N_PAD]
            tx2g = x2_ref[0:1, hi:N_PAD]
            ty2g = y2_ref[0:1, hi:N_PAD]
            tarea = (tx2g - tx1g) * (ty2g - ty1g)
            tx1 = jnp.maximum(cx1, tx1g)
            ty1 = jnp.maximum(cy1, ty1g)
            tx2 = jnp.minimum(kx2, tx2g)
            ty2 = jnp.minimum(cy2, ty2g)
            tw = jnp.maximum(tx2 - tx1, 0.0)
            th = jnp.maximum(ty2 - ty1, 0.0)
            tinter = tw * th
            tunion = areac + tarea - tinter
            tiou = tinter / jnp.maximum(tunion, 1e-9)
            supp = jnp.any(tiou > IOU_THR, axis=0, keepdims=True)
            alive_ref[0:1, hi:N_PAD] = (alive_ref[0:1, hi:N_PAD]
                                        * jnp.where(supp, 0.0, 1.0))

    alive = alive_ref[...]
    total_k = jnp.sum(alive)

    koff = jnp.float32(0.0)
    soff = jnp.float32(0.0)
    for k in range(NB):
        lo = k * BLK
        hi = lo + BLK
        row = alive_ref[0:1, lo:hi]
        kcum = jax.lax.dot_general(row, lti, (((1,), (0,)), ((), ())))
        nrow = 1.0 - row
        scum = jax.lax.dot_general(nrow, lti, (((1,), (0,)), ((), ())))
        dest_ref[0:1, lo:hi] = jnp.where(row > 0.5, koff + kcum - 1.0,
                                         total_k + soff + scum - 1.0)
        koff = koff + jnp.sum(row)
        soff = soff + jnp.sum(nrow)

    dest = dest_ref[...].astype(jnp.int32)
    tsub = jax.lax.broadcasted_iota(jnp.int32, (OUT_PAD, N_PAD), 0)
    m = jnp.where(dest == tsub, 1.0, 0.0).astype(_f32)

    def sel(row):
        return jax.lax.dot_general(m, row, (((1,), (1,)), ((), ())),
                                   precision=_HI)

    obox = jnp.concatenate([sel(x1_ref[...]), sel(y1_ref[...]),
                            sel(x2_ref[...]), sel(y2_ref[...])], axis=1)
    obox_ref[...] = obox
    smask = jnp.where(alive > 0.5, s, -1.0)
    os_ref[...] = jax.lax.dot_general(smask, m, (((1,), (1,)), ((), ())),
                                      precision=_HI)


def _run_nms(x1, y1, x2, y2, s, cx1, cy1, cx2, cy2):
    return pl.pallas_call(
        _nms_kernel,
        out_shape=[
            jax.ShapeDtypeStruct((OUT_PAD, 4), _f32),
            jax.ShapeDtypeStruct((1, OUT_PAD), _f32),
        ],
        scratch_shapes=[
            pltpu.VMEM((1, N_PAD), _f32),
            pltpu.VMEM((1, N_PAD), _f32),
        ],
    )(x1, y1, x2, y2, s, cx1, cy1, cx2, cy2)


def kernel(boxes, scores):
    pad = N_PAD - N_RAW
    nsp = jnp.concatenate([-scores, jnp.full((pad,), 3e38, _f32)])
    px1 = jnp.concatenate([boxes[:, 0], jnp.zeros((pad,), _f32)])
    py1 = jnp.concatenate([boxes[:, 1], jnp.zeros((pad,), _f32)])
    px2 = jnp.concatenate([boxes[:, 2], jnp.zeros((pad,), _f32)])
    py2 = jnp.concatenate([boxes[:, 3], jnp.zeros((pad,), _f32)])
    ns, sx1, sy1, sx2, sy2 = jax.lax.sort((nsp, px1, py1, px2, py2),
                                          num_keys=1)
    sp = (-ns).reshape(1, N_PAD)
    x1 = sx1.reshape(1, N_PAD)
    y1 = sy1.reshape(1, N_PAD)
    x2 = sx2.reshape(1, N_PAD)
    y2 = sy2.reshape(1, N_PAD)
    cx1 = sx1.reshape(N_PAD, 1)
    cy1 = sy1.reshape(N_PAD, 1)
    cx2 = sx2.reshape(N_PAD, 1)
    cy2 = sy2.reshape(N_PAD, 1)
    obox, ts = _run_nms(x1, y1, x2, y2, sp, cx1, cy1, cx2, cy2)
    return obox[:OUT_K], ts[0, :OUT_K]


# speedup vs baseline: 1.6396x; 1.0595x over previous
import jax
import jax.numpy as jnp
from jax.experimental import pallas as pl
from jax.experimental.pallas import tpu as pltpu

N_RAW = 5000
N_PAD = 5120
BLK = 128
NB = N_PAD // BLK
OUT_K = 300
OUT_PAD = 304
IOU_THR = 0.5
SCORE_THR = 0.05
FP_ITERS = 4
FP_CHUNK = 4

_HI = jax.lax.Precision.HIGHEST
_f32 = jnp.float32


def _nms_kernel(x1_ref, y1_ref, x2_ref, y2_ref, s_ref,
                cx1_ref, cy1_ref, cx2_ref, cy2_ref,
                obox_ref, os_ref, alive_ref, dest_ref):
    s = s_ref[...]

    sub = jax.lax.broadcasted_iota(jnp.int32, (BLK, BLK), 0)
    lan = jax.lax.broadcasted_iota(jnp.int32, (BLK, BLK), 1)
    eye = jnp.where(sub == lan, 1.0, 0.0).astype(_f32)
    lti = jnp.where(sub <= lan, 1.0, 0.0).astype(_f32)
    lowtri = lan < sub

    def tr(row):
        return jax.lax.dot_general(eye, row, (((1,), (1,)), ((), ())))

    valid = jnp.where(s > SCORE_THR, 1.0, 0.0).astype(_f32)

    def suppression_pass(exact):
        alive_ref[...] = valid
        acc = jnp.zeros((BLK, 1), _f32)
        for k in range(NB):
            lo = k * BLK
            hi = lo + BLK
            bx1 = x1_ref[0:1, lo:hi]
            by1 = y1_ref[0:1, lo:hi]
            bx2 = x2_ref[0:1, lo:hi]
            by2 = y2_ref[0:1, lo:hi]
            cx1 = cx1_ref[lo:hi, 0:1]
            cy1 = cy1_ref[lo:hi, 0:1]
            cx2 = cx2_ref[lo:hi, 0:1]
            cy2 = cy2_ref[lo:hi, 0:1]
            calive = tr(alive_ref[0:1, lo:hi])

            areac = (cx2 - cx1) * (cy2 - cy1)
            arear = (bx2 - bx1) * (by2 - by1)

            ix1 = jnp.maximum(cx1, bx1)
            iy1 = jnp.maximum(cy1, by1)
            ix2 = jnp.minimum(cx2, bx2)
            iy2 = jnp.minimum(cy2, by2)
            iw = jnp.maximum(ix2 - ix1, 0.0)
            ih = jnp.maximum(iy2 - iy1, 0.0)
            inter = iw * ih
            union = areac + arear - inter
            iou = inter / union
            sl = jnp.where((iou > IOU_THR) & lowtri, 1.0, 0.0).astype(_f32)

            def fp_step(keep, sl=sl, calive=calive):
                supp = jax.lax.dot_general(sl, keep,
                                           (((1,), (0,)), ((), ())))
                return calive * jnp.where(supp < 0.5, 1.0, 0.0)

            if exact:
                def fp_cond(c):
                    return c[1]

                def fp_body(c, fp_step=fp_step):
                    keep, _ = c
                    for _ in range(FP_CHUNK):
                        prev = keep
                        keep = fp_step(keep)
                    changed = jnp.sum(jnp.abs(keep - prev)) > 0.0
                    return (keep, changed)

                keepc, _ = jax.lax.while_loop(fp_cond, fp_body,
                                              (calive, jnp.array(True)))
            else:
                keepc = calive
                for _ in range(FP_ITERS):
                    prev = keepc
                    keepc = fp_step(keepc)
                acc = acc + jnp.abs(keepc - prev)

            keeprow = jax.lax.dot_general(keepc, eye,
                                          (((0,), (0,)), ((), ())))
            alive_ref[0:1, lo:hi] = keeprow

            if hi < N_PAD:
                kx2 = jnp.where(keepc > 0.5, cx2, -3e38)
                tx1g = x1_ref[0:1, hi:N_PAD]
                ty1g = y1_ref[0:1, hi:N_PAD]
                tx2g = x2_ref[0:1, hi:N_PAD]
                ty2g = y2_ref[0:1, hi:N_PAD]
                tarea = (tx2g - tx1g) * (ty2g - ty1g)
                tx1 = jnp.maximum(cx1, tx1g)
                ty1 = jnp.maximum(cy1, ty1g)
                tx2 = jnp.minimum(kx2, tx2g)
                ty2 = jnp.minimum(cy2, ty2g)
                tw = jnp.maximum(tx2 - tx1, 0.0)
                th = jnp.maximum(ty2 - ty1, 0.0)
                tinter = tw * th
                tunion = areac + tarea - tinter
                tiou = tinter / tunion
                supp = jnp.any(tiou > IOU_THR, axis=0, keepdims=True)
                alive_ref[0:1, hi:N_PAD] = (alive_ref[0:1, hi:N_PAD]
                                            * jnp.where(supp, 0.0, 1.0))
        return acc

    acc = suppression_pass(exact=False)
    unconverged = jnp.sum(acc) > 0.0

    @pl.when(unconverged)
    def _fallback():
        suppression_pass(exact=True)

    alive = alive_ref[...]
    total_k = jnp.sum(alive)

    koff = jnp.float32(0.0)
    soff = jnp.float32(0.0)
    for k in range(NB):
        lo = k * BLK
        hi = lo + BLK
        row = alive_ref[0:1, lo:hi]
        kcum = jax.lax.dot_general(row, lti, (((1,), (0,)), ((), ())))
        nrow = 1.0 - row
        scum = jax.lax.dot_general(nrow, lti, (((1,), (0,)), ((), ())))
        dest_ref[0:1, lo:hi] = jnp.where(row > 0.5, koff + kcum - 1.0,
                                         total_k + soff + scum - 1.0)
        koff = koff + jnp.sum(row)
        soff = soff + jnp.sum(nrow)

    dest = dest_ref[...].astype(jnp.int32)
    tsub = jax.lax.broadcasted_iota(jnp.int32, (OUT_PAD, N_PAD), 0)
    m = jnp.where(dest == tsub, 1.0, 0.0).astype(_f32)

    def sel(row):
        return jax.lax.dot_general(m, row, (((1,), (1,)), ((), ())),
                                   precision=_HI)

    obox = jnp.concatenate([sel(x1_ref[...]), sel(y1_ref[...]),
                            sel(x2_ref[...]), sel(y2_ref[...])], axis=1)
    obox_ref[...] = obox
    smask = jnp.where(alive > 0.5, s, -1.0)
    os_ref[...] = jax.lax.dot_general(smask, m, (((1,), (1,)), ((), ())),
                                      precision=_HI)


def _run_nms(x1, y1, x2, y2, s, cx1, cy1, cx2, cy2):
    return pl.pallas_call(
        _nms_kernel,
        out_shape=[
            jax.ShapeDtypeStruct((OUT_PAD, 4), _f32),
            jax.ShapeDtypeStruct((1, OUT_PAD), _f32),
        ],
        scratch_shapes=[
            pltpu.VMEM((1, N_PAD), _f32),
            pltpu.VMEM((1, N_PAD), _f32),
        ],
    )(x1, y1, x2, y2, s, cx1, cy1, cx2, cy2)


def kernel(boxes, scores):
    pad = N_PAD - N_RAW
    nsp = jnp.concatenate([-scores, jnp.full((pad,), 3e38, _f32)])
    px1 = jnp.concatenate([boxes[:, 0], jnp.zeros((pad,), _f32)])
    py1 = jnp.concatenate([boxes[:, 1], jnp.zeros((pad,), _f32)])
    px2 = jnp.concatenate([boxes[:, 2], jnp.zeros((pad,), _f32)])
    py2 = jnp.concatenate([boxes[:, 3], jnp.zeros((pad,), _f32)])
    ns, sx1, sy1, sx2, sy2 = jax.lax.sort((nsp, px1, py1, px2, py2),
                                          num_keys=1)
    sp = (-ns).reshape(1, N_PAD)
    x1 = sx1.reshape(1, N_PAD)
    y1 = sy1.reshape(1, N_PAD)
    x2 = sx2.reshape(1, N_PAD)
    y2 = sy2.reshape(1, N_PAD)
    cx1 = sx1.reshape(N_PAD, 1)
    cy1 = sy1.reshape(N_PAD, 1)
    cx2 = sx2.reshape(N_PAD, 1)
    cy2 = sy2.reshape(N_PAD, 1)
    obox, ts = _run_nms(x1, y1, x2, y2, sp, cx1, cy1, cx2, cy2)
    return obox[:OUT_K], ts[0, :OUT_K]


# speedup vs baseline: 1.6438x; 1.0025x over previous
import jax
import jax.numpy as jnp
from jax.experimental import pallas as pl
from jax.experimental.pallas import tpu as pltpu

N_RAW = 5000
N_PAD = 5120
BLK = 128
NB = N_PAD // BLK
OUT_K = 300
OUT_PAD = 304
IOU_THR = 0.5
SCORE_THR = 0.05
FP_ITERS = 4
FP_CHUNK = 4

_HI = jax.lax.Precision.HIGHEST
_f32 = jnp.float32


def _nms_kernel(x1_ref, y1_ref, x2_ref, y2_ref, s_ref,
                cx1_ref, cy1_ref, cx2_ref, cy2_ref,
                obox_ref, os_ref, alive_ref, dest_ref):
    s = s_ref[...]

    sub = jax.lax.broadcasted_iota(jnp.int32, (BLK, BLK), 0)
    lan = jax.lax.broadcasted_iota(jnp.int32, (BLK, BLK), 1)
    eye = jnp.where(sub == lan, 1.0, 0.0).astype(_f32)
    lti = jnp.where(sub <= lan, 1.0, 0.0).astype(_f32)
    lowtri = lan < sub

    def tr(row):
        return jax.lax.dot_general(eye, row, (((1,), (1,)), ((), ())))

    valid = jnp.where(s > SCORE_THR, 1.0, 0.0).astype(_f32)

    def suppression_pass(exact):
        alive_ref[...] = valid
        acc = jnp.zeros((BLK, 1), _f32)
        for k in range(NB):
            lo = k * BLK
            hi = lo + BLK
            bx1 = x1_ref[0:1, lo:hi]
            by1 = y1_ref[0:1, lo:hi]
            bx2 = x2_ref[0:1, lo:hi]
            by2 = y2_ref[0:1, lo:hi]
            cx1 = cx1_ref[lo:hi, 0:1]
            cy1 = cy1_ref[lo:hi, 0:1]
            cx2 = cx2_ref[lo:hi, 0:1]
            cy2 = cy2_ref[lo:hi, 0:1]
            calive = tr(alive_ref[0:1, lo:hi])

            areac = (cx2 - cx1) * (cy2 - cy1)
            arear = (bx2 - bx1) * (by2 - by1)

            ix1 = jnp.maximum(cx1, bx1)
            iy1 = jnp.maximum(cy1, by1)
            ix2 = jnp.minimum(cx2, bx2)
            iy2 = jnp.minimum(cy2, by2)
            iw = jnp.maximum(ix2 - ix1, 0.0)
            ih = jnp.maximum(iy2 - iy1, 0.0)
            inter = iw * ih
            union = areac + arear - inter
            iou = inter / union
            sl = jnp.where((iou > IOU_THR) & lowtri, 1.0, 0.0).astype(_f32)

            def fp_step(keep, sl=sl, calive=calive):
                supp = jax.lax.dot_general(sl, keep,
                                           (((1,), (0,)), ((), ())))
                return calive * jnp.where(supp < 0.5, 1.0, 0.0)

            if exact:
                def fp_cond(c):
                    return c[1]

                def fp_body(c, fp_step=fp_step):
                    keep, _ = c
                    for _ in range(FP_CHUNK):
                        prev = keep
                        keep = fp_step(keep)
                    changed = jnp.sum(jnp.abs(keep - prev)) > 0.0
                    return (keep, changed)

                keepc, _ = jax.lax.while_loop(fp_cond, fp_body,
                                              (calive, jnp.array(True)))
            else:
                keepc = calive
                for _ in range(FP_ITERS):
                    prev = keepc
                    keepc = fp_step(keepc)
                acc = acc + jnp.abs(keepc - prev)

            keeprow = jax.lax.dot_general(keepc, eye,
                                          (((0,), (0,)), ((), ())))
            alive_ref[0:1, lo:hi] = keeprow

            if hi < N_PAD:
                kx2 = jnp.where(keepc > 0.5, cx2, -3e38)
                tx1g = x1_ref[0:1, hi:N_PAD]
                ty1g = y1_ref[0:1, hi:N_PAD]
                tx2g = x2_ref[0:1, hi:N_PAD]
                ty2g = y2_ref[0:1, hi:N_PAD]
                tarea = (tx2g - tx1g) * (ty2g - ty1g)
                tx1 = jnp.maximum(cx1, tx1g)
                ty1 = jnp.maximum(cy1, ty1g)
                tx2 = jnp.minimum(kx2, tx2g)
                ty2 = jnp.minimum(cy2, ty2g)
                tw = jnp.maximum(tx2 - tx1, 0.0)
                th = jnp.maximum(ty2 - ty1, 0.0)
                tinter = tw * th
                tunion = areac + tarea - tinter
                tiou = tinter / tunion
                supp = jnp.any(tiou > IOU_THR, axis=0, keepdims=True)
                alive_ref[0:1, hi:N_PAD] = (alive_ref[0:1, hi:N_PAD]
                                            * jnp.where(supp, 0.0, 1.0))
        return acc

    acc = suppression_pass(exact=False)
    unconverged = jnp.sum(acc) > 0.0

    @pl.when(unconverged)
    def _fallback():
        suppression_pass(exact=True)

    alive = alive_ref[...]
    total_k = jnp.sum(alive, axis=1, keepdims=True)

    pos1 = (jax.lax.broadcasted_iota(jnp.int32, (1, BLK), 1)
            .astype(_f32) + 1.0)
    koff = jnp.zeros((1, 1), _f32)
    for k in range(NB):
        lo = k * BLK
        hi = lo + BLK
        row = alive_ref[0:1, lo:hi]
        kcum = jax.lax.dot_general(row, lti, (((1,), (0,)), ((), ())))
        gk = koff + kcum
        dest_ref[0:1, lo:hi] = jnp.where(
            row > 0.5, gk - 1.0, total_k + (lo + 0.0) + pos1 - gk - 1.0)
        koff = koff + jnp.sum(row, axis=1, keepdims=True)

    dest = dest_ref[...].astype(jnp.int32)
    tsub = jax.lax.broadcasted_iota(jnp.int32, (OUT_PAD, N_PAD), 0)
    m = jnp.where(dest == tsub, 1.0, 0.0).astype(_f32)

    def sel(row):
        return jax.lax.dot_general(m, row, (((1,), (1,)), ((), ())),
                                   precision=_HI)

    obox = jnp.concatenate([sel(x1_ref[...]), sel(y1_ref[...]),
                            sel(x2_ref[...]), sel(y2_ref[...])], axis=1)
    obox_ref[...] = obox
    smask = jnp.where(alive > 0.5, s, -1.0)
    os_ref[...] = jax.lax.dot_general(smask, m, (((1,), (1,)), ((), ())),
                                      precision=_HI)


def _run_nms(x1, y1, x2, y2, s, cx1, cy1, cx2, cy2):
    return pl.pallas_call(
        _nms_kernel,
        out_shape=[
            jax.ShapeDtypeStruct((OUT_PAD, 4), _f32),
            jax.ShapeDtypeStruct((1, OUT_PAD), _f32),
        ],
        scratch_shapes=[
            pltpu.VMEM((1, N_PAD), _f32),
            pltpu.VMEM((1, N_PAD), _f32),
        ],
    )(x1, y1, x2, y2, s, cx1, cy1, cx2, cy2)


def kernel(boxes, scores):
    pad = N_PAD - N_RAW
    nsp = jnp.concatenate([-scores, jnp.full((pad,), 3e38, _f32)])
    px1 = jnp.concatenate([boxes[:, 0], jnp.zeros((pad,), _f32)])
    py1 = jnp.concatenate([boxes[:, 1], jnp.zeros((pad,), _f32)])
    px2 = jnp.concatenate([boxes[:, 2], jnp.zeros((pad,), _f32)])
    py2 = jnp.concatenate([boxes[:, 3], jnp.zeros((pad,), _f32)])
    ns, sx1, sy1, sx2, sy2 = jax.lax.sort((nsp, px1, py1, px2, py2),
                                          num_keys=1)
    sp = (-ns).reshape(1, N_PAD)
    x1 = sx1.reshape(1, N_PAD)
    y1 = sy1.reshape(1, N_PAD)
    x2 = sx2.reshape(1, N_PAD)
    y2 = sy2.reshape(1, N_PAD)
    cx1 = sx1.reshape(N_PAD, 1)
    cy1 = sy1.reshape(N_PAD, 1)
    cx2 = sx2.reshape(N_PAD, 1)
    cy2 = sy2.reshape(N_PAD, 1)
    obox, ts = _run_nms(x1, y1, x2, y2, sp, cx1, cy1, cx2, cy2)
    return obox[:OUT_K], ts[0, :OUT_K]
